# Initial kernel scaffold; baseline (speedup 1.0000x reference)
import jax
import jax.numpy as jnp
from jax.experimental import pallas as pl


def kernel(x):
    raise NotImplementedError("write your pallas kernel here")


# speedup vs baseline: 1.7662x; 1.7662x over previous
import jax
import jax.numpy as jnp
from jax.experimental import pallas as pl

_NUM_CLASS = 1000


def _onehot_body(x_ref, o_ref):
    x = x_ref[...]
    shape = (x.shape[0], x.shape[1], _NUM_CLASS)
    iota = jax.lax.broadcasted_iota(jnp.int32, shape, 2)
    o_ref[...] = (iota == x[:, :, None]).astype(jnp.float32)


def kernel(x):
    B, L = x.shape
    x = x.astype(jnp.int32)
    Bb = 64
    out = pl.pallas_call(
        _onehot_body,
        grid=(B // Bb,),
        in_specs=[pl.BlockSpec((Bb, L), lambda i: (i, 0))],
        out_specs=pl.BlockSpec((Bb, L, _NUM_CLASS), lambda i: (i, 0, 0)),
        out_shape=jax.ShapeDtypeStruct((B, L, _NUM_CLASS), jnp.float32),
    )(x)
    return out
